# baseline (speedup 1.0000x reference)
import jax
import jax.numpy as jnp
import numpy as np
from jax.experimental import pallas as pl

BB = 128
BF = jnp.bfloat16
F32 = jnp.float32


def _roll_rows(x, up):
    if up:
        return jnp.concatenate([x[1:], x[:1]], axis=0)
    return jnp.concatenate([x[-1:], x[:-1]], axis=0)


def _mask(hpad, nrows, lanes, i_lo, i_hi, j_lo, j_hi, cblk=128):
    r = jax.lax.broadcasted_iota(jnp.int32, (nrows, 1), 0)
    ip = jax.lax.rem(r, hpad)
    row_ok = jnp.logical_and(ip >= i_lo, ip < i_hi)
    l = jax.lax.broadcasted_iota(jnp.int32, (1, lanes), 1)
    jp = jax.lax.div(l, cblk)
    col_ok = jnp.logical_and(jp >= j_lo, jp < j_hi)
    return jnp.logical_and(row_ok, col_ok)


def _conv_1x3_blocks(xc, xm, xp, w_ref, hw_pad, nctr):
    w0 = w_ref[0].astype(BF)
    w1 = w_ref[1].astype(BF)
    w2 = w_ref[2].astype(BF)
    nrows = xc.shape[0]
    zero = jnp.zeros((nrows, 128), F32)
    blocks = [zero]
    for j in range(1, nctr + 1):
        lo, hi = (j - 1) * 128, (j + 2) * 128
        acc = jnp.dot(xm[:, lo:hi], w0, preferred_element_type=F32)
        acc = acc + jnp.dot(xc[:, lo:hi], w1, preferred_element_type=F32)
        acc = acc + jnp.dot(xp[:, lo:hi], w2, preferred_element_type=F32)
        blocks.append(acc)
    blocks.append(zero)
    return jnp.concatenate(blocks, axis=1)


def _kernel(d3_ref, w1z_ref, b1t_ref, w2r_ref, b2t_ref, w3r_ref, b3t_ref,
            w4r_ref, b4t_ref, w5_ref, b5_ref, w6_ref, b6_ref,
            outx_ref, outy_ref):
    d3 = d3_ref[...]

    y1 = jnp.dot(d3.astype(BF), w1z_ref[...].astype(BF),
                 preferred_element_type=F32)
    m12 = _mask(12, BB * 12, 12 * 128, 1, 11, 1, 11)
    x1 = jnp.where(m12, jax.nn.relu(y1 + b1t_ref[0]), 0.0)

    x1b = x1.astype(BF)
    x1m = _roll_rows(x1b, up=False)
    x1p = _roll_rows(x1b, up=True)
    y2 = _conv_1x3_blocks(x1b, x1m, x1p, w2r_ref, 12, 10)
    x2 = jnp.where(m12, jax.nn.relu(y2 + b2t_ref[0]), 0.0)

    x2v = x2.reshape(BB, 12, 12 * 128)
    rows = []
    zrow = jnp.zeros((BB, 1, 7 * 128), F32)
    rows.append(zrow)
    for k in range(5):
        pr = jnp.maximum(x2v[:, 1 + 2 * k, :], x2v[:, 2 + 2 * k, :])
        blks = [jnp.zeros((BB, 128), F32)]
        for m in range(5):
            lo1, lo2 = (1 + 2 * m) * 128, (2 + 2 * m) * 128
            blks.append(jnp.maximum(pr[:, lo1:lo1 + 128], pr[:, lo2:lo2 + 128]))
        blks.append(jnp.zeros((BB, 128), F32))
        rows.append(jnp.concatenate(blks, axis=1).reshape(BB, 1, 7 * 128))
    rows.append(zrow)
    p1 = jnp.concatenate(rows, axis=1).reshape(BB * 7, 7 * 128)

    m7 = _mask(7, BB * 7, 7 * 128, 1, 6, 1, 6)
    p1b = p1.astype(BF)
    y3 = _conv_1x3_blocks(p1b, _roll_rows(p1b, False), _roll_rows(p1b, True),
                          w3r_ref, 7, 5)
    x3 = jnp.where(m7, jax.nn.relu(y3 + b3t_ref[0]), 0.0)

    x3b = x3.astype(BF)
    y4 = _conv_1x3_blocks(x3b, _roll_rows(x3b, False), _roll_rows(x3b, True),
                          w4r_ref, 7, 5)
    x4 = jnp.where(m7, jax.nn.relu(y4 + b4t_ref[0]), 0.0)

    x4v = x4.reshape(BB, 7, 7 * 128)
    pr0 = jnp.maximum(x4v[:, 1, :], x4v[:, 2, :])
    pr1 = jnp.maximum(x4v[:, 3, :], x4v[:, 4, :])
    qs = []
    for pr in (pr0, pr1):
        for m in range(2):
            lo1, lo2 = (1 + 2 * m) * 128, (2 + 2 * m) * 128
            qs.append(jnp.maximum(pr[:, lo1:lo1 + 128], pr[:, lo2:lo2 + 128]))
    g = (qs[0] + qs[1] + qs[2] + qs[3]) * 0.25

    h = jax.nn.relu(jnp.dot(g[:, :64].astype(BF), w5_ref[...].astype(BF),
                            preferred_element_type=F32) + b5_ref[0])
    svg = jax.nn.sigmoid(jnp.dot(h.astype(BF), w6_ref[...].astype(BF),
                                 preferred_element_type=F32) + b6_ref[0])

    p1x, p1y = svg[:, 0:1], svg[:, 1:2]
    p2x, p2y = svg[:, 2:3], svg[:, 3:4]
    ts = [0.0, 0.25, 0.5, 0.75, 1.0]
    px = jnp.concatenate([(1.0 - t) * p1x + t * p2x for t in ts], axis=1)
    py = jnp.concatenate([(1.0 - t) * p1y + t * p2y for t in ts], axis=1)
    px = jnp.round(px * 10.0)
    py = jnp.round(py * 10.0)

    pos = px * 10.0 + py

    rank_cols = []
    for i in range(5):
        pi = pos[:, i:i + 1]
        lt = (pos < pi).astype(F32)
        r = jnp.sum(lt, axis=1, keepdims=True)
        if i:
            r = r + jnp.sum((pos[:, :i] == pi).astype(F32), axis=1,
                            keepdims=True)
        rank_cols.append(r)
    rank = jnp.concatenate(rank_cols, axis=1)

    sx_cols, sy_cols = [], []
    for k in range(5):
        m = (rank == float(k)).astype(F32)
        sx_cols.append(jnp.sum(m * px, axis=1, keepdims=True))
        sy_cols.append(jnp.sum(m * py, axis=1, keepdims=True))
    sx = jnp.concatenate(sx_cols, axis=1)
    sy = jnp.concatenate(sy_cols, axis=1)

    diffs = (jnp.abs(sx[:, 1:] - sx[:, :-1]) +
             jnp.abs(sy[:, 1:] - sy[:, :-1]))
    mask = jnp.concatenate(
        [jnp.ones((BB, 1), F32), (diffs != 0.0).astype(F32)], axis=1)
    counts = jnp.sum(mask, axis=1, keepdims=True)

    run = jnp.zeros((BB, 1), F32)
    pre_cols = []
    for i in range(5):
        pre_cols.append(run)
        run = run + mask[:, i:i + 1]
    pre_kept = jnp.concatenate(pre_cols, axis=1)
    iota_row = jax.lax.broadcasted_iota(jnp.int32, (BB, 5), 1).astype(F32)
    rank2 = jnp.where(mask > 0.5, pre_kept, counts + iota_row - pre_kept)

    ox_cols, oy_cols = [], []
    for k in range(5):
        m = (rank2 == float(k)).astype(F32)
        v = (float(k) < counts)
        ox_cols.append(jnp.where(v, jnp.sum(m * sx, axis=1, keepdims=True), -1.0))
        oy_cols.append(jnp.where(v, jnp.sum(m * sy, axis=1, keepdims=True), -1.0))

    outx_ref[...] = jnp.concatenate(ox_cols, axis=1)
    outy_ref[...] = jnp.concatenate(oy_cols, axis=1)


def _prep_weights(W1, b1, W2, b2, W3, b3, W4, b4):
    w1 = W1[:, :, 0, :]
    z = jnp.zeros((3, 12, 12, 128), F32)
    for jp in range(1, 11):
        for dx in range(3):
            z = z.at[:, jp - 1 + dx, jp, :].set(w1[:, dx, :])
    W1z = z.reshape(36, 12 * 128)
    b1t = jnp.tile(b1, 12).reshape(1, -1)

    W2r = W2.reshape(3, 384, 128)
    b2t = jnp.tile(b2, 12).reshape(1, -1)

    W3r = jnp.pad(W3, ((0, 0), (0, 0), (0, 0), (0, 64))).reshape(3, 384, 128)
    b3t = jnp.tile(jnp.pad(b3, (0, 64)), 7).reshape(1, -1)

    W4p = jnp.pad(W4, ((0, 0), (0, 0), (0, 64), (0, 64)))
    W4r = W4p.reshape(3, 384, 128)
    b4t = jnp.tile(jnp.pad(b4, (0, 64)), 7).reshape(1, -1)
    return W1z, b1t, W2r, b2t, W3r, b3t, W4r, b4t


@jax.jit
def _run(data, W1, b1, W2, b2, W3, b3, W4, b4, W5, b5, W6, b6):
    B = data.shape[0]
    grid = (B // BB,)

    W1z, b1t, W2r, b2t, W3r, b3t, W4r, b4t = _prep_weights(
        W1, b1, W2, b2, W3, b3, W4, b4)
    b5r, b6r = b5.reshape(1, -1), b6.reshape(1, -1)

    dp = jnp.pad(data[..., 0], ((0, 0), (1, 1), (1, 1)))
    up = jnp.roll(dp, -1, axis=1)
    dn = jnp.roll(dp, 1, axis=1)
    d3 = jnp.concatenate([dn, dp, up], axis=2).reshape(B * 12, 36)

    def bspec(shape):
        return pl.BlockSpec(shape, lambda i: (0,) * len(shape))

    return pl.pallas_call(
        _kernel,
        grid=grid,
        in_specs=[
            pl.BlockSpec((BB * 12, 36), lambda i: (i, 0)),
            bspec(W1z.shape), bspec(b1t.shape),
            bspec(W2r.shape), bspec(b2t.shape),
            bspec(W3r.shape), bspec(b3t.shape),
            bspec(W4r.shape), bspec(b4t.shape),
            bspec(W5.shape), bspec(b5r.shape),
            bspec(W6.shape), bspec(b6r.shape),
        ],
        out_specs=[pl.BlockSpec((BB, 5), lambda i: (i, 0)),
                   pl.BlockSpec((BB, 5), lambda i: (i, 0))],
        out_shape=[jax.ShapeDtypeStruct((B, 5), F32),
                   jax.ShapeDtypeStruct((B, 5), F32)],
    )(d3, W1z, b1t, W2r, b2t, W3r, b3t, W4r, b4t, W5, b5r, W6, b6r)


def kernel(data, W1, b1, W2, b2, W3, b3, W4, b4, W5, b5, W6, b6):
    ox, oy = _run(data, W1, b1, W2, b2, W3, b3, W4, b4, W5, b5, W6, b6)
    return jnp.stack([ox, oy], axis=2)


# speedup vs baseline: 1.0044x; 1.0044x over previous
import jax
import jax.numpy as jnp
import numpy as np
from jax.experimental import pallas as pl

BB = 128
BF = jnp.bfloat16
F32 = jnp.float32


def _roll_rows(x, up):
    if up:
        return jnp.concatenate([x[1:], x[:1]], axis=0)
    return jnp.concatenate([x[-1:], x[:-1]], axis=0)


def _row_mask(hpad, nrows, i_lo, i_hi):
    r = jax.lax.broadcasted_iota(jnp.int32, (nrows, 1), 0)
    ip = jax.lax.rem(r, hpad)
    return jnp.logical_and(ip >= i_lo, ip < i_hi)


def _conv_1x3_blocks(xc, xm, xp, w_ref, hw_pad, nctr):
    w0 = w_ref[0].astype(BF)
    w1 = w_ref[1].astype(BF)
    w2 = w_ref[2].astype(BF)
    nrows = xc.shape[0]
    zero = jnp.zeros((nrows, 128), F32)
    blocks = [zero]
    for j in range(1, nctr + 1):
        lo, hi = (j - 1) * 128, (j + 2) * 128
        acc = jnp.dot(xm[:, lo:hi], w0, preferred_element_type=F32)
        acc = acc + jnp.dot(xc[:, lo:hi], w1, preferred_element_type=F32)
        acc = acc + jnp.dot(xp[:, lo:hi], w2, preferred_element_type=F32)
        blocks.append(acc)
    blocks.append(zero)
    return jnp.concatenate(blocks, axis=1)


def _kernel(d3_ref, w1z_ref, b1t_ref, w2r_ref, b2t_ref, w3r_ref, b3t_ref,
            w4r_ref, b4t_ref, w5_ref, b5_ref, w6_ref, b6_ref,
            outx_ref, outy_ref):
    d3 = d3_ref[...]

    y1 = jnp.dot(d3.astype(BF), w1z_ref[...].astype(BF),
                 preferred_element_type=F32)
    m12 = _row_mask(12, BB * 12, 1, 11)
    x1b = jnp.where(m12, jax.nn.relu(y1 + b1t_ref[0]), 0.0).astype(BF)

    x1m = _roll_rows(x1b, up=False)
    x1p = _roll_rows(x1b, up=True)
    y2 = _conv_1x3_blocks(x1b, x1m, x1p, w2r_ref, 12, 10)
    x2 = jnp.where(m12, jax.nn.relu(y2 + b2t_ref[0]), 0.0).astype(BF)

    t = jnp.maximum(x2, _roll_rows(x2, up=True))
    u = jnp.maximum(t[:, :-128], t[:, 128:])
    uv = u.reshape(BB, 12, 11 * 128)
    zrow = jnp.zeros((BB, 1, 7 * 128), BF)
    zblk = jnp.zeros((BB, 128), BF)
    rows = [zrow]
    for k in range(5):
        pr = uv[:, 1 + 2 * k, :]
        blks = [zblk]
        for m in range(5):
            lo = (1 + 2 * m) * 128
            blks.append(pr[:, lo:lo + 128])
        blks.append(zblk)
        rows.append(jnp.concatenate(blks, axis=1).reshape(BB, 1, 7 * 128))
    rows.append(zrow)
    p1b = jnp.concatenate(rows, axis=1).reshape(BB * 7, 7 * 128)

    m7 = _row_mask(7, BB * 7, 1, 6)
    y3 = _conv_1x3_blocks(p1b, _roll_rows(p1b, False), _roll_rows(p1b, True),
                          w3r_ref, 7, 5)
    x3b = jnp.where(m7, jax.nn.relu(y3 + b3t_ref[0]), 0.0).astype(BF)

    y4 = _conv_1x3_blocks(x3b, _roll_rows(x3b, False), _roll_rows(x3b, True),
                          w4r_ref, 7, 5)
    x4 = jnp.where(m7, jax.nn.relu(y4 + b4t_ref[0]), 0.0)

    t4 = jnp.maximum(x4, _roll_rows(x4, up=True))
    u4 = jnp.maximum(t4[:, :-128], t4[:, 128:])
    u4v = u4.reshape(BB, 7, 6 * 128)
    qs = []
    for k in range(2):
        pr = u4v[:, 1 + 2 * k, :]
        for m in range(2):
            lo = (1 + 2 * m) * 128
            qs.append(pr[:, lo:lo + 128])
    g = (qs[0] + qs[1] + qs[2] + qs[3]) * 0.25

    h = jax.nn.relu(jnp.dot(g[:, :64].astype(BF), w5_ref[...].astype(BF),
                            preferred_element_type=F32) + b5_ref[0])
    svg = jax.nn.sigmoid(jnp.dot(h.astype(BF), w6_ref[...].astype(BF),
                                 preferred_element_type=F32) + b6_ref[0])

    p1x, p1y = svg[:, 0:1], svg[:, 1:2]
    p2x, p2y = svg[:, 2:3], svg[:, 3:4]
    ts = [0.0, 0.25, 0.5, 0.75, 1.0]
    px = jnp.concatenate([(1.0 - t) * p1x + t * p2x for t in ts], axis=1)
    py = jnp.concatenate([(1.0 - t) * p1y + t * p2y for t in ts], axis=1)
    px = jnp.round(px * 10.0)
    py = jnp.round(py * 10.0)

    pos = px * 10.0 + py

    rank_cols = []
    for i in range(5):
        pi = pos[:, i:i + 1]
        lt = (pos < pi).astype(F32)
        r = jnp.sum(lt, axis=1, keepdims=True)
        if i:
            r = r + jnp.sum((pos[:, :i] == pi).astype(F32), axis=1,
                            keepdims=True)
        rank_cols.append(r)
    rank = jnp.concatenate(rank_cols, axis=1)

    sx_cols, sy_cols = [], []
    for k in range(5):
        m = (rank == float(k)).astype(F32)
        sx_cols.append(jnp.sum(m * px, axis=1, keepdims=True))
        sy_cols.append(jnp.sum(m * py, axis=1, keepdims=True))
    sx = jnp.concatenate(sx_cols, axis=1)
    sy = jnp.concatenate(sy_cols, axis=1)

    diffs = (jnp.abs(sx[:, 1:] - sx[:, :-1]) +
             jnp.abs(sy[:, 1:] - sy[:, :-1]))
    mask = jnp.concatenate(
        [jnp.ones((BB, 1), F32), (diffs != 0.0).astype(F32)], axis=1)
    counts = jnp.sum(mask, axis=1, keepdims=True)

    run = jnp.zeros((BB, 1), F32)
    pre_cols = []
    for i in range(5):
        pre_cols.append(run)
        run = run + mask[:, i:i + 1]
    pre_kept = jnp.concatenate(pre_cols, axis=1)
    iota_row = jax.lax.broadcasted_iota(jnp.int32, (BB, 5), 1).astype(F32)
    rank2 = jnp.where(mask > 0.5, pre_kept, counts + iota_row - pre_kept)

    ox_cols, oy_cols = [], []
    for k in range(5):
        m = (rank2 == float(k)).astype(F32)
        v = (float(k) < counts)
        ox_cols.append(jnp.where(v, jnp.sum(m * sx, axis=1, keepdims=True), -1.0))
        oy_cols.append(jnp.where(v, jnp.sum(m * sy, axis=1, keepdims=True), -1.0))

    outx_ref[...] = jnp.concatenate(ox_cols, axis=1)
    outy_ref[...] = jnp.concatenate(oy_cols, axis=1)


def _prep_weights(W1, b1, W2, b2, W3, b3, W4, b4):
    w1 = W1[:, :, 0, :]
    z = jnp.zeros((3, 12, 12, 128), F32)
    for jp in range(1, 11):
        for dx in range(3):
            z = z.at[:, jp - 1 + dx, jp, :].set(w1[:, dx, :])
    W1z = z.reshape(36, 12 * 128)

    def tile_ctr(b, nblk):
        t = jnp.tile(b, nblk - 2)
        z128 = jnp.zeros((128,), F32)
        return jnp.concatenate([z128, t, z128]).reshape(1, -1)

    b1t = tile_ctr(b1, 12)
    W2r = W2.reshape(3, 384, 128)
    b2t = tile_ctr(b2, 12)

    W3r = jnp.pad(W3, ((0, 0), (0, 0), (0, 0), (0, 64))).reshape(3, 384, 128)
    b3t = tile_ctr(jnp.pad(b3, (0, 64)), 7)

    W4p = jnp.pad(W4, ((0, 0), (0, 0), (0, 64), (0, 64)))
    W4r = W4p.reshape(3, 384, 128)
    b4t = tile_ctr(jnp.pad(b4, (0, 64)), 7)
    return W1z, b1t, W2r, b2t, W3r, b3t, W4r, b4t


@jax.jit
def _run(data, W1, b1, W2, b2, W3, b3, W4, b4, W5, b5, W6, b6):
    B = data.shape[0]
    grid = (B // BB,)

    W1z, b1t, W2r, b2t, W3r, b3t, W4r, b4t = _prep_weights(
        W1, b1, W2, b2, W3, b3, W4, b4)
    b5r, b6r = b5.reshape(1, -1), b6.reshape(1, -1)

    dp = jnp.pad(data[..., 0], ((0, 0), (1, 1), (1, 1)))
    up = jnp.roll(dp, -1, axis=1)
    dn = jnp.roll(dp, 1, axis=1)
    d3 = jnp.concatenate([dn, dp, up], axis=2).reshape(B * 12, 36)

    def bspec(shape):
        return pl.BlockSpec(shape, lambda i: (0,) * len(shape))

    return pl.pallas_call(
        _kernel,
        grid=grid,
        in_specs=[
            pl.BlockSpec((BB * 12, 36), lambda i: (i, 0)),
            bspec(W1z.shape), bspec(b1t.shape),
            bspec(W2r.shape), bspec(b2t.shape),
            bspec(W3r.shape), bspec(b3t.shape),
            bspec(W4r.shape), bspec(b4t.shape),
            bspec(W5.shape), bspec(b5r.shape),
            bspec(W6.shape), bspec(b6r.shape),
        ],
        out_specs=[pl.BlockSpec((BB, 5), lambda i: (i, 0)),
                   pl.BlockSpec((BB, 5), lambda i: (i, 0))],
        out_shape=[jax.ShapeDtypeStruct((B, 5), F32),
                   jax.ShapeDtypeStruct((B, 5), F32)],
    )(d3, W1z, b1t, W2r, b2t, W3r, b3t, W4r, b4t, W5, b5r, W6, b6r)


def kernel(data, W1, b1, W2, b2, W3, b3, W4, b4, W5, b5, W6, b6):
    ox, oy = _run(data, W1, b1, W2, b2, W3, b3, W4, b4, W5, b5, W6, b6)
    return jnp.stack([ox, oy], axis=2)
